# baseline (speedup 1.0000x reference)
import jax
import jax.numpy as jnp
from jax import lax
from jax.experimental import pallas as pl
from jax.experimental.pallas import tpu as pltpu

ROWS = 128
COLS = 32768
BS = 8192
SUB = BS // 128
GRID = COLS // BS
I32_MAX = 2**31 - 1


def _tc_body(x_ref, out_ref, amax_ref, astep_ref):
    j = pl.program_id(0)

    @pl.when(j == 0)
    def _init():
        amax_ref[...] = jnp.full((ROWS, 128), -jnp.inf, jnp.float32)
        astep_ref[...] = jnp.zeros((ROWS, 128), jnp.int32)

    amax = amax_ref[...]
    astep = astep_ref[...]
    for s in range(SUB):
        v = x_ref[:, s * 128:(s + 1) * 128]
        step = j * SUB + s
        take = v > amax
        amax = jnp.where(take, v, amax)
        astep = jnp.where(take, step, astep)
    amax_ref[...] = amax
    astep_ref[...] = astep

    @pl.when(j == GRID - 1)
    def _finish():
        lanes = lax.broadcasted_iota(jnp.int32, (ROWS, 128), 1)
        idx = astep * 128 + lanes
        gmax = jnp.max(amax, axis=1, keepdims=True)
        cand = jnp.where(amax == gmax, idx, I32_MAX)
        out_ref[...] = jnp.min(cand, axis=1)


_argmax_tc = pl.pallas_call(
    _tc_body,
    grid=(GRID,),
    in_specs=[pl.BlockSpec((ROWS, BS), lambda j: (0, j))],
    out_specs=pl.BlockSpec((ROWS,), lambda j: (0,)),
    out_shape=jax.ShapeDtypeStruct((ROWS,), jnp.int32),
    scratch_shapes=[
        pltpu.VMEM((ROWS, 128), jnp.float32),
        pltpu.VMEM((ROWS, 128), jnp.int32),
    ],
)


@jax.jit
def kernel(x):
    return _argmax_tc(x)
